# Initial kernel scaffold; baseline (speedup 1.0000x reference)
import jax
import jax.numpy as jnp
from jax.experimental import pallas as pl


def kernel(X, edge_index, W1, b1, u1, W2, b2, u2, Wg, bg, Wl, bl, gn_weight, gn_bias, gn_mean_scale, Wo, bo):
    raise NotImplementedError("write your pallas kernel here")


# speedup vs baseline: 38.7701x; 38.7701x over previous
import functools

import jax
import jax.numpy as jnp
from jax import lax
from jax.experimental import pallas as pl
from jax.experimental.pallas import tpu as pltpu
from jax.experimental.pallas import tpu_sc as plsc

_N = 2048
_L = 240
_L2 = 236
_C = 256
_E = 65536
_B = 4
_NB = 256
_QROWS = 512
_QWORDS = _QROWS * _N


def _sc_count_body(dst_ref, src_ref, a_ref, ebuf_d, ebuf_s, idx1, val1, zbuf,
                   shared):
    cid = lax.axis_index("c")
    sid = lax.axis_index("s")
    zeros16 = jnp.zeros((16,), jnp.float32)
    ones16 = jnp.ones((16,), jnp.float32)
    iota16 = lax.iota(jnp.int32, 16)

    def _zi(i, c):
        zbuf[pl.ds(i * 16, 16)] = zeros16
        return c
    lax.fori_loop(0, 256, _zi, 0)

    eb = sid * 4096
    pltpu.sync_copy(dst_ref.at[pl.ds(eb, 4096)], ebuf_d)
    pltpu.sync_copy(src_ref.at[pl.ds(eb, 4096)], ebuf_s)

    for p in range(2):
        q = cid * 2 + p
        qbase = q * _QROWS
        for r in range(16):
            pltpu.sync_copy(zbuf, shared.at[pl.ds(sid * 65536 + r * 4096, 4096)])
        plsc.subcore_barrier()

        def _chunk(j, c):
            for jj in range(8):
                off = j * 128 + jj * 16
                dv = ebuf_d[pl.ds(off, 16)]
                sv = ebuf_s[pl.ds(off, 16)]
                rel = dv - qbase
                m = (rel >= 0) & (rel < _QROWS)
                idx = (rel << 11) + sv
                idx = jnp.where(m, idx, iota16 * 2048 + sid * 16)
                val = jnp.where(m, ones16, zeros16)
                idx1[pl.ds(jj * 16, 16)] = idx
                val1[pl.ds(jj * 16, 16)] = val
            pltpu.sync_copy(val1, shared.at[idx1], add=True)
            return c
        lax.fori_loop(0, 32, _chunk, 0)
        plsc.subcore_barrier()

        pltpu.sync_copy(shared.at[pl.ds(sid * 65536, 65536)],
                        a_ref.at[pl.ds(qbase * 2048 + sid * 65536, 65536)])
        plsc.subcore_barrier()


def _build_counts(dst, src):
    mesh = plsc.VectorSubcoreMesh(core_axis_name="c", subcore_axis_name="s")
    f = pl.kernel(
        _sc_count_body,
        mesh=mesh,
        out_type=jax.ShapeDtypeStruct((_N * _N,), jnp.float32),
        scratch_types=[
            pltpu.VMEM((4096,), jnp.int32),
            pltpu.VMEM((4096,), jnp.int32),
            pltpu.VMEM((128,), jnp.int32),
            pltpu.VMEM((128,), jnp.float32),
            pltpu.VMEM((4096,), jnp.float32),
            pltpu.VMEM_SHARED((_QWORDS,), jnp.float32),
        ],
    )
    return f(dst, src)


def _sn_body(wt1_ref, u1_ref, wt2_ref, u2_ref, s1_ref, s2_ref):
    def sig(wt, u):
        v = jnp.sum(wt * u, axis=1, keepdims=True)
        nv = jnp.sqrt(jnp.sum(v * v))
        vn = v / (nv + 1e-12)
        w = jnp.sum(wt * vn, axis=0, keepdims=True)
        nw = jnp.sqrt(jnp.sum(w * w))
        return (nw * nw) / (nw + 1e-12)

    s1_ref[...] = sig(wt1_ref[...], u1_ref[...]).reshape(1, 1)
    s2_ref[...] = sig(wt2_ref[...], u2_ref[...]).reshape(1, 1)


def _sn_call(wt1, u1r, wt2, u2r):
    return pl.pallas_call(
        _sn_body,
        out_shape=[jax.ShapeDtypeStruct((1, 1), jnp.float32),
                   jax.ShapeDtypeStruct((1, 1), jnp.float32)],
    )(wt1, u1r, wt2, u2r)


def _deg_body(a_ref, dinv_ref):
    deg = jnp.sum(a_ref[...], axis=1, keepdims=True) + 1.0
    dinv_ref[...] = lax.rsqrt(deg)


def _deg_call(a):
    return pl.pallas_call(
        _deg_body,
        grid=(_N // _NB,),
        in_specs=[pl.BlockSpec((_NB, _N), lambda i: (i, 0))],
        out_specs=pl.BlockSpec((_NB, 1), lambda i: (i, 0)),
        out_shape=jax.ShapeDtypeStruct((_N, 1), jnp.float32),
    )(a)


def _conv_body(x_ref, w1_ref, b1_ref, w2_ref, b2_ref, wg_ref, wl_ref, bl_ref,
               s1_ref, s2_ref, h_ref, r_ref):
    x = x_ref[0]
    w1 = w1_ref[...] * (1.0 / s1_ref[0, 0])
    w2 = w2_ref[...] * (1.0 / s2_ref[0, 0])
    b1 = b1_ref[...]
    acc = b2_ref[...] + jnp.zeros((_NB, _L2), jnp.float32)
    for f in range(16):
        s = (x[:, 0:238] * w1[:, 3 * f:3 * f + 1] +
             x[:, 1:239] * w1[:, 3 * f + 1:3 * f + 2] +
             x[:, 2:240] * w1[:, 3 * f + 2:3 * f + 3] +
             b1[:, f:f + 1])
        y = jnp.maximum(s, 0.01 * s)
        acc = (acc +
               y[:, 0:236] * w2[:, 3 * f:3 * f + 1] +
               y[:, 1:237] * w2[:, 3 * f + 1:3 * f + 2] +
               y[:, 2:238] * w2[:, 3 * f + 2:3 * f + 3])
    c2 = jnp.maximum(acc, 0.01 * acc)
    h_ref[0] = jnp.dot(c2, wg_ref[...], preferred_element_type=jnp.float32)
    r_ref[0] = (jnp.dot(x, wl_ref[...], preferred_element_type=jnp.float32)
                + bl_ref[...])


def _conv_call(X, w1r, b1r, w2r, b2r, Wg, Wl, blr, s1, s2):
    nblocks = _N // _NB
    return pl.pallas_call(
        _conv_body,
        grid=(_B, nblocks),
        in_specs=[
            pl.BlockSpec((1, _NB, _L), lambda b, i: (b, i, 0)),
            pl.BlockSpec((_NB, 48), lambda b, i: (i, 0)),
            pl.BlockSpec((_NB, 16), lambda b, i: (i, 0)),
            pl.BlockSpec((_NB, 48), lambda b, i: (i, 0)),
            pl.BlockSpec((_NB, 1), lambda b, i: (i, 0)),
            pl.BlockSpec((_L2, _C), lambda b, i: (0, 0)),
            pl.BlockSpec((_L, _C), lambda b, i: (0, 0)),
            pl.BlockSpec((1, _C), lambda b, i: (0, 0)),
            pl.BlockSpec((1, 1), lambda b, i: (0, 0)),
            pl.BlockSpec((1, 1), lambda b, i: (0, 0)),
        ],
        out_specs=[
            pl.BlockSpec((1, _NB, _C), lambda b, i: (b, i, 0)),
            pl.BlockSpec((1, _NB, _C), lambda b, i: (b, i, 0)),
        ],
        out_shape=[jax.ShapeDtypeStruct((_B, _N, _C), jnp.float32),
                   jax.ShapeDtypeStruct((_B, _N, _C), jnp.float32)],
    )(X, w1r, b1r, w2r, b2r, Wg, Wl, blr, s1, s2)


def _gcn_body(a_ref, hp_ref, hblk_ref, r_ref, dcol_ref, drow_ref, bg_ref,
              gnw_ref, gnb_ref, gms_ref, wo_ref, bo_ref, out_ref, cls_ref):
    i = pl.program_id(0)
    nsteps = pl.num_programs(0)
    a = a_ref[...] * drow_ref[...]
    dloc = dcol_ref[...]
    bg = bg_ref[...]
    ts = []
    for c in range(_B):
        m = jnp.dot(a, hp_ref[c], preferred_element_type=jnp.float32)
        t = dloc * m + (dloc * dloc) * hblk_ref[c] + bg
        ts.append(t)
    mean = (ts[0] + ts[1] + ts[2] + ts[3]) * 0.25
    gms = gms_ref[...]
    outs = [t - mean * gms for t in ts]
    var = (outs[0] * outs[0] + outs[1] * outs[1] +
           outs[2] * outs[2] + outs[3] * outs[3]) * 0.25
    denom = lax.rsqrt(var + 1e-5)
    gnw = gnw_ref[...]
    gnb = gnb_ref[...]
    psums = []
    for c in range(_B):
        tt = gnw * outs[c] * denom + gnb + r_ref[c]
        tt = jnp.where(tt > 0, tt, jnp.exp(tt) - 1.0)
        psums.append(jnp.sum(tt, axis=0, keepdims=True))
    add = jnp.concatenate(psums, axis=0)

    @pl.when(i == 0)
    def _():
        out_ref[...] = jnp.zeros_like(out_ref)

    out_ref[...] += add

    @pl.when(i == nsteps - 1)
    def _():
        out = out_ref[...] * (1.0 / _N)
        out_ref[...] = out
        cls_ref[...] = (jnp.dot(out, wo_ref[...],
                                preferred_element_type=jnp.float32)
                        + bo_ref[...])


def _gcn_call(a, h, r, dcol, drow, bgr, gnwr, gnbr, gmsr, Wo, bor):
    nblocks = _N // _NB
    return pl.pallas_call(
        _gcn_body,
        grid=(nblocks,),
        in_specs=[
            pl.BlockSpec((_NB, _N), lambda i: (i, 0)),
            pl.BlockSpec((_B, _N, _C), lambda i: (0, 0, 0)),
            pl.BlockSpec((_B, _NB, _C), lambda i: (0, i, 0)),
            pl.BlockSpec((_B, _NB, _C), lambda i: (0, i, 0)),
            pl.BlockSpec((_NB, 1), lambda i: (i, 0)),
            pl.BlockSpec((1, _N), lambda i: (0, 0)),
            pl.BlockSpec((1, _C), lambda i: (0, 0)),
            pl.BlockSpec((1, _C), lambda i: (0, 0)),
            pl.BlockSpec((1, _C), lambda i: (0, 0)),
            pl.BlockSpec((1, _C), lambda i: (0, 0)),
            pl.BlockSpec((_C, 16), lambda i: (0, 0)),
            pl.BlockSpec((1, 16), lambda i: (0, 0)),
        ],
        out_specs=[
            pl.BlockSpec((_B, _C), lambda i: (0, 0)),
            pl.BlockSpec((_B, 16), lambda i: (0, 0)),
        ],
        out_shape=[jax.ShapeDtypeStruct((_B, _C), jnp.float32),
                   jax.ShapeDtypeStruct((_B, 16), jnp.float32)],
    )(a, h, h, r, dcol, drow, bgr, gnwr, gnbr, gmsr, Wo, bor)


def kernel(X, edge_index, W1, b1, u1, W2, b2, u2, Wg, bg, Wl, bl,
           gn_weight, gn_bias, gn_mean_scale, Wo, bo):
    src = edge_index[0]
    dst = edge_index[1]
    a_flat = _build_counts(dst, src)
    a = a_flat.reshape(_N, _N)

    wt1 = W1.reshape(16 * _N, 3).T
    u1r = u1.reshape(1, 16 * _N)
    wt2 = W2.reshape(_N, 48).T
    u2r = u2.reshape(1, _N)
    s1, s2 = _sn_call(wt1, u1r, wt2, u2r)

    dcol = _deg_call(a)
    drow = dcol.reshape(1, _N)

    w1r = W1.reshape(_N, 48)
    b1r = b1.reshape(_N, 16)
    w2r = W2.reshape(_N, 48)
    b2r = b2.reshape(_N, 1)
    blr = bl.reshape(1, _C)
    h, r = _conv_call(X, w1r, b1r, w2r, b2r, Wg, Wl, blr, s1, s2)

    output, cls = _gcn_call(a, h, r, dcol, drow, bg.reshape(1, _C),
                            gn_weight.reshape(1, _C), gn_bias.reshape(1, _C),
                            gn_mean_scale.reshape(1, _C), Wo,
                            bo.reshape(1, 16))
    return (output, cls)


# speedup vs baseline: 92.0266x; 2.3736x over previous
import functools

import jax
import jax.numpy as jnp
from jax import lax
from jax.experimental import pallas as pl
from jax.experimental.pallas import tpu as pltpu
from jax.experimental.pallas import tpu_sc as plsc

_N = 2048
_L = 240
_L2 = 236
_C = 256
_E = 65536
_B = 4
_NB = 256
_QROWS = 512
_QWORDS = _QROWS * _N


def _sc_count_body(dst_ref, src_ref, a_ref, ebuf_d, ebuf_s, idx1, val1, zbuf,
                   shared):
    cid = lax.axis_index("c")
    sid = lax.axis_index("s")
    zeros16 = jnp.zeros((16,), jnp.float32)
    ones16 = jnp.ones((16,), jnp.float32)
    iota16 = lax.iota(jnp.int32, 16)

    def _zi(i, c):
        zbuf[pl.ds(i * 16, 16)] = zeros16
        return c
    lax.fori_loop(0, 256, _zi, 0)

    eb = sid * 4096
    pltpu.sync_copy(dst_ref.at[pl.ds(eb, 4096)], ebuf_d)
    pltpu.sync_copy(src_ref.at[pl.ds(eb, 4096)], ebuf_s)

    for p in range(2):
        q = cid * 2 + p
        qbase = q * _QROWS
        for r in range(16):
            pltpu.sync_copy(zbuf, shared.at[pl.ds(sid * 65536 + r * 4096, 4096)])
        plsc.subcore_barrier()

        def _chunk(j, c):
            for jj in range(8):
                off = j * 128 + jj * 16
                dv = ebuf_d[pl.ds(off, 16)]
                sv = ebuf_s[pl.ds(off, 16)]
                rel = dv - qbase
                m = (rel >= 0) & (rel < _QROWS)
                idx = (rel << 11) + sv
                idx = jnp.where(m, idx, iota16 * 2048 + sid * 16)
                val = jnp.where(m, ones16, zeros16)
                idx1[pl.ds(jj * 16, 16)] = idx
                val1[pl.ds(jj * 16, 16)] = val
            pltpu.sync_copy(val1, shared.at[idx1], add=True)
            return c
        lax.fori_loop(0, 32, _chunk, 0)
        plsc.subcore_barrier()

        pltpu.sync_copy(shared.at[pl.ds(sid * 65536, 65536)],
                        a_ref.at[pl.ds(qbase * 2048 + sid * 65536, 65536)])
        plsc.subcore_barrier()


def _build_counts(dst, src):
    mesh = plsc.VectorSubcoreMesh(core_axis_name="c", subcore_axis_name="s")
    f = pl.kernel(
        _sc_count_body,
        mesh=mesh,
        out_type=jax.ShapeDtypeStruct((_N * _N,), jnp.float32),
        scratch_types=[
            pltpu.VMEM((4096,), jnp.int32),
            pltpu.VMEM((4096,), jnp.int32),
            pltpu.VMEM((128,), jnp.int32),
            pltpu.VMEM((128,), jnp.float32),
            pltpu.VMEM((4096,), jnp.float32),
            pltpu.VMEM_SHARED((_QWORDS,), jnp.float32),
        ],
    )
    return f(dst, src)


def _sn_body(wt1_ref, u1_ref, wt2_ref, u2_ref, s1_ref, s2_ref):
    def sig(wt, u):
        v = jnp.sum(wt * u, axis=1, keepdims=True)
        nv = jnp.sqrt(jnp.sum(v * v))
        vn = v / (nv + 1e-12)
        w = jnp.sum(wt * vn, axis=0, keepdims=True)
        nw = jnp.sqrt(jnp.sum(w * w))
        return (nw * nw) / (nw + 1e-12)

    s1_ref[...] = sig(wt1_ref[...], u1_ref[...]).reshape(1, 1)
    s2_ref[...] = sig(wt2_ref[...], u2_ref[...]).reshape(1, 1)


def _sn_call(wt1, u1r, wt2, u2r):
    return pl.pallas_call(
        _sn_body,
        out_shape=[jax.ShapeDtypeStruct((1, 1), jnp.float32),
                   jax.ShapeDtypeStruct((1, 1), jnp.float32)],
    )(wt1, u1r, wt2, u2r)


def _deg_body(a_ref, dinv_ref):
    deg = jnp.sum(a_ref[...], axis=1, keepdims=True) + 1.0
    dinv_ref[...] = lax.rsqrt(deg)


def _deg_call(a):
    return pl.pallas_call(
        _deg_body,
        grid=(_N // _NB,),
        in_specs=[pl.BlockSpec((_NB, _N), lambda i: (i, 0))],
        out_specs=pl.BlockSpec((_NB, 1), lambda i: (i, 0)),
        out_shape=jax.ShapeDtypeStruct((_N, 1), jnp.float32),
    )(a)


_NBW = 512


def _conv_body(x_ref, w1_ref, b1_ref, w2_ref, b2_ref, wg_ref, wl_ref, bl_ref,
               s1_ref, s2_ref, h_ref, r_ref):
    x = x_ref[0]
    w1 = w1_ref[...] * (1.0 / s1_ref[0, 0])
    w2 = w2_ref[...] * (1.0 / s2_ref[0, 0])
    b1 = b1_ref[...]
    acc = b2_ref[...] + jnp.zeros((_L2, _NBW), jnp.float32)
    x0 = x[0:238]
    x1 = x[1:239]
    x2 = x[2:240]
    for f in range(16):
        s = (x0 * w1[3 * f:3 * f + 1] +
             x1 * w1[3 * f + 1:3 * f + 2] +
             x2 * w1[3 * f + 2:3 * f + 3] +
             b1[f:f + 1])
        y = jnp.maximum(s, 0.01 * s)
        acc = (acc +
               y[0:236] * w2[3 * f:3 * f + 1] +
               y[1:237] * w2[3 * f + 1:3 * f + 2] +
               y[2:238] * w2[3 * f + 2:3 * f + 3])
    c2 = jnp.maximum(acc, 0.01 * acc)
    h_ref[0] = lax.dot_general(c2, wg_ref[...], (((0,), (0,)), ((), ())),
                               preferred_element_type=jnp.float32)
    r_ref[0] = (lax.dot_general(x, wl_ref[...], (((0,), (0,)), ((), ())),
                                preferred_element_type=jnp.float32)
                + bl_ref[...])


def _conv_call(Xt, w1t, b1t, w2t, b2t, Wg, Wl, blr, s1, s2):
    nblocks = _N // _NBW
    return pl.pallas_call(
        _conv_body,
        grid=(_B, nblocks),
        in_specs=[
            pl.BlockSpec((1, _L, _NBW), lambda b, i: (b, 0, i)),
            pl.BlockSpec((48, _NBW), lambda b, i: (0, i)),
            pl.BlockSpec((16, _NBW), lambda b, i: (0, i)),
            pl.BlockSpec((48, _NBW), lambda b, i: (0, i)),
            pl.BlockSpec((1, _NBW), lambda b, i: (0, i)),
            pl.BlockSpec((_L2, _C), lambda b, i: (0, 0)),
            pl.BlockSpec((_L, _C), lambda b, i: (0, 0)),
            pl.BlockSpec((1, _C), lambda b, i: (0, 0)),
            pl.BlockSpec((1, 1), lambda b, i: (0, 0)),
            pl.BlockSpec((1, 1), lambda b, i: (0, 0)),
        ],
        out_specs=[
            pl.BlockSpec((1, _NBW, _C), lambda b, i: (b, i, 0)),
            pl.BlockSpec((1, _NBW, _C), lambda b, i: (b, i, 0)),
        ],
        out_shape=[jax.ShapeDtypeStruct((_B, _N, _C), jnp.float32),
                   jax.ShapeDtypeStruct((_B, _N, _C), jnp.float32)],
    )(Xt, w1t, b1t, w2t, b2t, Wg, Wl, blr, s1, s2)


def _gcn_body(a_ref, hp_ref, hblk_ref, r_ref, dcol_ref, drow_ref, bg_ref,
              gnw_ref, gnb_ref, gms_ref, wo_ref, bo_ref, out_ref, cls_ref):
    i = pl.program_id(0)
    nsteps = pl.num_programs(0)
    a = a_ref[...] * drow_ref[...]
    dloc = dcol_ref[...]
    bg = bg_ref[...]
    ts = []
    for c in range(_B):
        m = jnp.dot(a, hp_ref[c], preferred_element_type=jnp.float32)
        t = dloc * m + (dloc * dloc) * hblk_ref[c] + bg
        ts.append(t)
    mean = (ts[0] + ts[1] + ts[2] + ts[3]) * 0.25
    gms = gms_ref[...]
    outs = [t - mean * gms for t in ts]
    var = (outs[0] * outs[0] + outs[1] * outs[1] +
           outs[2] * outs[2] + outs[3] * outs[3]) * 0.25
    denom = lax.rsqrt(var + 1e-5)
    gnw = gnw_ref[...]
    gnb = gnb_ref[...]
    psums = []
    for c in range(_B):
        tt = gnw * outs[c] * denom + gnb + r_ref[c]
        tt = jnp.where(tt > 0, tt, jnp.exp(tt) - 1.0)
        psums.append(jnp.sum(tt, axis=0, keepdims=True))
    add = jnp.concatenate(psums, axis=0)

    @pl.when(i == 0)
    def _():
        out_ref[...] = jnp.zeros_like(out_ref)

    out_ref[...] += add

    @pl.when(i == nsteps - 1)
    def _():
        out = out_ref[...] * (1.0 / _N)
        out_ref[...] = out
        cls_ref[...] = (jnp.dot(out, wo_ref[...],
                                preferred_element_type=jnp.float32)
                        + bo_ref[...])


def _gcn_call(a, h, r, dcol, drow, bgr, gnwr, gnbr, gmsr, Wo, bor):
    nblocks = _N // _NB
    return pl.pallas_call(
        _gcn_body,
        grid=(nblocks,),
        in_specs=[
            pl.BlockSpec((_NB, _N), lambda i: (i, 0)),
            pl.BlockSpec((_B, _N, _C), lambda i: (0, 0, 0)),
            pl.BlockSpec((_B, _NB, _C), lambda i: (0, i, 0)),
            pl.BlockSpec((_B, _NB, _C), lambda i: (0, i, 0)),
            pl.BlockSpec((_NB, 1), lambda i: (i, 0)),
            pl.BlockSpec((1, _N), lambda i: (0, 0)),
            pl.BlockSpec((1, _C), lambda i: (0, 0)),
            pl.BlockSpec((1, _C), lambda i: (0, 0)),
            pl.BlockSpec((1, _C), lambda i: (0, 0)),
            pl.BlockSpec((1, _C), lambda i: (0, 0)),
            pl.BlockSpec((_C, 16), lambda i: (0, 0)),
            pl.BlockSpec((1, 16), lambda i: (0, 0)),
        ],
        out_specs=[
            pl.BlockSpec((_B, _C), lambda i: (0, 0)),
            pl.BlockSpec((_B, 16), lambda i: (0, 0)),
        ],
        out_shape=[jax.ShapeDtypeStruct((_B, _C), jnp.float32),
                   jax.ShapeDtypeStruct((_B, 16), jnp.float32)],
    )(a, h, h, r, dcol, drow, bgr, gnwr, gnbr, gmsr, Wo, bor)


def kernel(X, edge_index, W1, b1, u1, W2, b2, u2, Wg, bg, Wl, bl,
           gn_weight, gn_bias, gn_mean_scale, Wo, bo):
    src = edge_index[0]
    dst = edge_index[1]
    a_flat = _build_counts(dst, src)
    a = a_flat.reshape(_N, _N)

    wt1 = W1.reshape(16 * _N, 3).T
    u1r = u1.reshape(1, 16 * _N)
    wt2 = W2.reshape(_N, 48).T
    u2r = u2.reshape(1, _N)
    s1, s2 = _sn_call(wt1, u1r, wt2, u2r)

    dcol = _deg_call(a)
    drow = dcol.reshape(1, _N)

    Xt = X.transpose(0, 2, 1)
    w1t = W1.reshape(_N, 16, 3).transpose(1, 2, 0).reshape(48, _N)
    b1t = b1.reshape(_N, 16).T
    w2t = W2.transpose(1, 2, 0).reshape(48, _N)
    b2t = b2.reshape(1, _N)
    blr = bl.reshape(1, _C)
    h, r = _conv_call(Xt, w1t, b1t, w2t, b2t, Wg, Wl, blr, s1, s2)

    output, cls = _gcn_call(a, h, r, dcol, drow, bg.reshape(1, _C),
                            gn_weight.reshape(1, _C), gn_bias.reshape(1, _C),
                            gn_mean_scale.reshape(1, _C), Wo,
                            bo.reshape(1, 16))
    return (output, cls)
